# baseline (speedup 1.0000x reference)
import jax
import jax.numpy as jnp
from jax import lax
from jax.experimental import pallas as pl
from jax.experimental.pallas import tpu as pltpu
from jax.experimental.pallas import tpu_sc as plsc

N = 2048
E = 4096
NC = 2
NS = 16
NW = NC * NS
DPW = N // NW
TAB = 101 * 101
TABP = 10208
THRESHOLD = 10.0
CHUNK = 4


def _sc_body(exf_hbm, eyf_hbm, cx_hbm, cy_hbm, t0_hbm, t1_hbm, tn_hbm,
             s0_hbm, s1_hbm, dec_hbm,
             ex_v, ey_v, t0_v, t1_v, tn_v, cxl_v, cyl_v, o0_v, o1_v, od_v):
    cid = lax.axis_index("c")
    sid = lax.axis_index("s")
    wid = sid * NC + cid
    base = wid * DPW

    pltpu.sync_copy(exf_hbm, ex_v)
    pltpu.sync_copy(eyf_hbm, ey_v)
    pltpu.sync_copy(t0_hbm, t0_v)
    pltpu.sync_copy(t1_hbm, t1_v)
    pltpu.sync_copy(tn_hbm, tn_v)
    pltpu.sync_copy(cx_hbm.at[pl.ds(base, DPW)], cxl_v)
    pltpu.sync_copy(cy_hbm.at[pl.ds(base, DPW)], cyl_v)

    for g in range(DPW // 16):
        cxv = cxl_v[pl.ds(g * 16, 16)]
        cyv = cyl_v[pl.ds(g * 16, 16)]

        z = jnp.zeros((16,), jnp.float32)

        @plsc.parallel_loop(0, E // CHUNK, carry=(z, z, z), unroll=4)
        def chunk_step(c, carry, cxv=cxv, cyv=cyv):
            s0, s1, cn = carry
            ebase = jnp.zeros((16,), jnp.int32) + c * CHUNK
            for j in range(CHUNK):
                idx = ebase + j
                exs = plsc.load_gather(ex_v, [idx])
                eys = plsc.load_gather(ey_v, [idx])
                dxf = jnp.clip(exs - cxv, -50.0, 50.0)
                dyf = jnp.clip(eys - cyv, -50.0, 50.0)
                dxi = dxf.astype(jnp.int32)
                dyi = dyf.astype(jnp.int32)
                f = dxi * 101 + (dyi + (50 * 101 + 50))
                s0 = s0 + plsc.load_gather(t0_v, [f])
                s1 = s1 + plsc.load_gather(t1_v, [f])
                cn = cn + plsc.load_gather(tn_v, [f])
            return (s0, s1, cn)

        s0, s1, cn = chunk_step
        o0_v[pl.ds(g * 16, 16)] = s0
        o1_v[pl.ds(g * 16, 16)] = s1
        od_v[pl.ds(g * 16, 16)] = jnp.where(cn >= THRESHOLD, 1.0, 0.0).astype(jnp.float32)

    pltpu.sync_copy(o0_v, s0_hbm.at[pl.ds(base, DPW)])
    pltpu.sync_copy(o1_v, s1_hbm.at[pl.ds(base, DPW)])
    pltpu.sync_copy(od_v, dec_hbm.at[pl.ds(base, DPW)])


_sc_gather = pl.kernel(
    _sc_body,
    out_type=(
        jax.ShapeDtypeStruct((N,), jnp.float32),
        jax.ShapeDtypeStruct((N,), jnp.float32),
        jax.ShapeDtypeStruct((N,), jnp.float32),
    ),
    mesh=plsc.VectorSubcoreMesh(core_axis_name="c", subcore_axis_name="s"),
    compiler_params=pltpu.CompilerParams(needs_layout_passes=False),
    scratch_types=[
        pltpu.VMEM((E,), jnp.float32),
        pltpu.VMEM((E,), jnp.float32),
        pltpu.VMEM((TABP,), jnp.float32),
        pltpu.VMEM((TABP,), jnp.float32),
        pltpu.VMEM((TABP,), jnp.float32),
        pltpu.VMEM((DPW,), jnp.float32),
        pltpu.VMEM((DPW,), jnp.float32),
        pltpu.VMEM((DPW,), jnp.float32),
        pltpu.VMEM((DPW,), jnp.float32),
        pltpu.VMEM((DPW,), jnp.float32),
    ],
)

BR = 256


def _tc_pair_body(cc, mask, pd, cxr, cyr, corr, cdx, cdy):
    cyrow = cc[:, 0:1]
    cxrow = cc[:, 1:2]
    dxc = cxr[0:1, :] - cxrow
    dyc = cyr[0:1, :] - cyrow
    m = mask[...]
    sdx = dxc * m
    sdy = dyc * m
    p = pd[...]
    radi = sdx * sdx + sdy * sdy - p * p
    stx = jnp.sum(4.0 * dxc * radi, axis=1, keepdims=True)
    sty = jnp.sum(4.0 * dyc * radi, axis=1, keepdims=True)
    cdx[...] = corr[...] * stx
    cdy[...] = corr[...] * sty


_tc_pair = pl.pallas_call(
    _tc_pair_body,
    grid=(N // BR,),
    in_specs=[
        pl.BlockSpec((BR, 2), lambda i: (i, 0)),
        pl.BlockSpec((BR, N), lambda i: (i, 0)),
        pl.BlockSpec((BR, N), lambda i: (i, 0)),
        pl.BlockSpec((1, N), lambda i: (0, 0)),
        pl.BlockSpec((1, N), lambda i: (0, 0)),
        pl.BlockSpec((BR, 1), lambda i: (i, 0)),
    ],
    out_specs=[
        pl.BlockSpec((BR, 1), lambda i: (i, 0)),
        pl.BlockSpec((BR, 1), lambda i: (i, 0)),
    ],
    out_shape=[
        jax.ShapeDtypeStruct((N, 1), jnp.float32),
        jax.ShapeDtypeStruct((N, 1), jnp.float32),
    ],
)


def _tc_combine_body(cc, s0, s1, dec, cdx, cdy, ny, nx):
    d = dec[...]
    ux = jnp.clip(s0[...], -400.0, 400.0)
    uy = jnp.clip(s1[...], -400.0, 400.0)
    nx[...] = cc[:, 1:2] - 200 * 1.5e-05 * d * (ux - 1.0 * 2.5e-07 * cdx[...])
    ny[...] = cc[:, 0:1] - 200 * 1.5e-05 * d * (uy - 1.0 * 2.5e-07 * cdy[...])


_tc_combine = pl.pallas_call(
    _tc_combine_body,
    out_shape=[
        jax.ShapeDtypeStruct((N, 1), jnp.float32),
        jax.ShapeDtypeStruct((N, 1), jnp.float32),
    ],
)


@jax.jit
def kernel(events_x, events_y, calib_center, precompute_grid,
           pairwise_dists_mask, pairwise_dists, correction):
    exf = events_x.astype(jnp.float32)
    eyf = events_y.astype(jnp.float32)
    cx = calib_center[:, 1]
    cy = calib_center[:, 0]
    g0 = precompute_grid[:, :, 0].reshape(-1)
    g1 = precompute_grid[:, :, 1].reshape(-1)
    gn = (g0 != 0).astype(jnp.float32) + (g1 != 0).astype(jnp.float32)
    pad = TABP - TAB
    t0 = jnp.pad(g0, (0, pad))
    t1 = jnp.pad(g1, (0, pad))
    tn = jnp.pad(gn, (0, pad))
    s0, s1, dec = _sc_gather(exf, eyf, cx, cy, t0, t1, tn)
    cdx, cdy = _tc_pair(calib_center, pairwise_dists_mask, pairwise_dists,
                        cx.reshape(1, N), cy.reshape(1, N),
                        correction.reshape(N, 1))
    ny, nx = _tc_combine(calib_center, s0.reshape(N, 1), s1.reshape(N, 1),
                         dec.reshape(N, 1), cdx, cdy)
    return jnp.concatenate([ny, nx], axis=1)


# speedup vs baseline: 1.0579x; 1.0579x over previous
import jax
import jax.numpy as jnp
from jax import lax
from jax.experimental import pallas as pl
from jax.experimental.pallas import tpu as pltpu
from jax.experimental.pallas import tpu_sc as plsc

N = 2048
E = 4096
NC = 2
NS = 16
NW = NC * NS
DPW = N // NW
TAB = 101 * 101
TABP = 10208
THRESHOLD = 10.0
CHUNK = 2


def _sc_body(exf_hbm, eyf_hbm, cx_hbm, cy_hbm, t0_hbm, t1_hbm, tn_hbm,
             s0_hbm, s1_hbm, dec_hbm,
             ex_v, ey_v, t0_v, t1_v, tn_v, cxl_v, cyl_v, o0_v, o1_v, od_v):
    cid = lax.axis_index("c")
    sid = lax.axis_index("s")
    wid = sid * NC + cid
    base = wid * DPW

    pltpu.sync_copy(exf_hbm, ex_v)
    pltpu.sync_copy(eyf_hbm, ey_v)
    pltpu.sync_copy(t0_hbm, t0_v)
    pltpu.sync_copy(t1_hbm, t1_v)
    pltpu.sync_copy(tn_hbm, tn_v)
    pltpu.sync_copy(cx_hbm.at[pl.ds(base, DPW)], cxl_v)
    pltpu.sync_copy(cy_hbm.at[pl.ds(base, DPW)], cyl_v)

    for g in range(DPW // 16):
        cxv = cxl_v[pl.ds(g * 16, 16)]
        cyv = cyl_v[pl.ds(g * 16, 16)]

        z = jnp.zeros((16,), jnp.float32)

        @plsc.parallel_loop(0, E // CHUNK, carry=(z, z, z), unroll=2)
        def chunk_step(c, carry, cxv=cxv, cyv=cyv):
            s0, s1, cn = carry
            ebase = jnp.zeros((16,), jnp.int32) + c * CHUNK
            for j in range(CHUNK):
                idx = ebase + j
                exs = plsc.load_gather(ex_v, [idx])
                eys = plsc.load_gather(ey_v, [idx])
                dxf = jnp.clip(exs - cxv, -50.0, 50.0)
                dyf = jnp.clip(eys - cyv, -50.0, 50.0)
                dxi = dxf.astype(jnp.int32)
                dyi = dyf.astype(jnp.int32)
                f = dxi * 101 + (dyi + (50 * 101 + 50))
                s0 = s0 + plsc.load_gather(t0_v, [f])
                s1 = s1 + plsc.load_gather(t1_v, [f])
                cn = cn + plsc.load_gather(tn_v, [f])
            return (s0, s1, cn)

        s0, s1, cn = chunk_step
        o0_v[pl.ds(g * 16, 16)] = s0
        o1_v[pl.ds(g * 16, 16)] = s1
        od_v[pl.ds(g * 16, 16)] = jnp.where(cn >= THRESHOLD, 1.0, 0.0).astype(jnp.float32)

    pltpu.sync_copy(o0_v, s0_hbm.at[pl.ds(base, DPW)])
    pltpu.sync_copy(o1_v, s1_hbm.at[pl.ds(base, DPW)])
    pltpu.sync_copy(od_v, dec_hbm.at[pl.ds(base, DPW)])


_sc_gather = pl.kernel(
    _sc_body,
    out_type=(
        jax.ShapeDtypeStruct((N,), jnp.float32),
        jax.ShapeDtypeStruct((N,), jnp.float32),
        jax.ShapeDtypeStruct((N,), jnp.float32),
    ),
    mesh=plsc.VectorSubcoreMesh(core_axis_name="c", subcore_axis_name="s"),
    compiler_params=pltpu.CompilerParams(needs_layout_passes=False),
    scratch_types=[
        pltpu.VMEM((E,), jnp.float32),
        pltpu.VMEM((E,), jnp.float32),
        pltpu.VMEM((TABP,), jnp.float32),
        pltpu.VMEM((TABP,), jnp.float32),
        pltpu.VMEM((TABP,), jnp.float32),
        pltpu.VMEM((DPW,), jnp.float32),
        pltpu.VMEM((DPW,), jnp.float32),
        pltpu.VMEM((DPW,), jnp.float32),
        pltpu.VMEM((DPW,), jnp.float32),
        pltpu.VMEM((DPW,), jnp.float32),
    ],
)

BR = 256


def _tc_pair_body(cc, mask, pd, cxr, cyr, corr, cdx, cdy):
    cyrow = cc[:, 0:1]
    cxrow = cc[:, 1:2]
    dxc = cxr[0:1, :] - cxrow
    dyc = cyr[0:1, :] - cyrow
    m = mask[...]
    sdx = dxc * m
    sdy = dyc * m
    p = pd[...]
    radi = sdx * sdx + sdy * sdy - p * p
    stx = jnp.sum(4.0 * dxc * radi, axis=1, keepdims=True)
    sty = jnp.sum(4.0 * dyc * radi, axis=1, keepdims=True)
    cdx[...] = corr[...] * stx
    cdy[...] = corr[...] * sty


_tc_pair = pl.pallas_call(
    _tc_pair_body,
    grid=(N // BR,),
    in_specs=[
        pl.BlockSpec((BR, 2), lambda i: (i, 0)),
        pl.BlockSpec((BR, N), lambda i: (i, 0)),
        pl.BlockSpec((BR, N), lambda i: (i, 0)),
        pl.BlockSpec((1, N), lambda i: (0, 0)),
        pl.BlockSpec((1, N), lambda i: (0, 0)),
        pl.BlockSpec((BR, 1), lambda i: (i, 0)),
    ],
    out_specs=[
        pl.BlockSpec((BR, 1), lambda i: (i, 0)),
        pl.BlockSpec((BR, 1), lambda i: (i, 0)),
    ],
    out_shape=[
        jax.ShapeDtypeStruct((N, 1), jnp.float32),
        jax.ShapeDtypeStruct((N, 1), jnp.float32),
    ],
)


def _tc_combine_body(cc, s0, s1, dec, cdx, cdy, ny, nx):
    d = dec[...]
    ux = jnp.clip(s0[...], -400.0, 400.0)
    uy = jnp.clip(s1[...], -400.0, 400.0)
    nx[...] = cc[:, 1:2] - 200 * 1.5e-05 * d * (ux - 1.0 * 2.5e-07 * cdx[...])
    ny[...] = cc[:, 0:1] - 200 * 1.5e-05 * d * (uy - 1.0 * 2.5e-07 * cdy[...])


_tc_combine = pl.pallas_call(
    _tc_combine_body,
    out_shape=[
        jax.ShapeDtypeStruct((N, 1), jnp.float32),
        jax.ShapeDtypeStruct((N, 1), jnp.float32),
    ],
)


@jax.jit
def kernel(events_x, events_y, calib_center, precompute_grid,
           pairwise_dists_mask, pairwise_dists, correction):
    exf = events_x.astype(jnp.float32)
    eyf = events_y.astype(jnp.float32)
    cx = calib_center[:, 1]
    cy = calib_center[:, 0]
    g0 = precompute_grid[:, :, 0].reshape(-1)
    g1 = precompute_grid[:, :, 1].reshape(-1)
    gn = (g0 != 0).astype(jnp.float32) + (g1 != 0).astype(jnp.float32)
    pad = TABP - TAB
    t0 = jnp.pad(g0, (0, pad))
    t1 = jnp.pad(g1, (0, pad))
    tn = jnp.pad(gn, (0, pad))
    s0, s1, dec = _sc_gather(exf, eyf, cx, cy, t0, t1, tn)
    cdx, cdy = _tc_pair(calib_center, pairwise_dists_mask, pairwise_dists,
                        cx.reshape(1, N), cy.reshape(1, N),
                        correction.reshape(N, 1))
    ny, nx = _tc_combine(calib_center, s0.reshape(N, 1), s1.reshape(N, 1),
                         dec.reshape(N, 1), cdx, cdy)
    return jnp.concatenate([ny, nx], axis=1)


# speedup vs baseline: 1.1002x; 1.0401x over previous
import jax
import jax.numpy as jnp
from jax import lax
from jax.experimental import pallas as pl
from jax.experimental.pallas import tpu as pltpu
from jax.experimental.pallas import tpu_sc as plsc

N = 2048
E = 4096
NC = 2
NS = 16
NW = NC * NS
DPW = N // NW
TAB = 101 * 101
TABP = 10208
THRESHOLD = 10.0
CHUNK = 4


def _sc_body(exf_hbm, eyf_hbm, cx_hbm, cy_hbm, t0_hbm, t1_hbm, tn_hbm,
             s0_hbm, s1_hbm, dec_hbm,
             ex_v, ey_v, t0_v, t1_v, tn_v, cxl_v, cyl_v, o0_v, o1_v, od_v):
    cid = lax.axis_index("c")
    sid = lax.axis_index("s")
    wid = sid * NC + cid
    base = wid * DPW

    pltpu.sync_copy(exf_hbm, ex_v)
    pltpu.sync_copy(eyf_hbm, ey_v)
    pltpu.sync_copy(t0_hbm, t0_v)
    pltpu.sync_copy(t1_hbm, t1_v)
    pltpu.sync_copy(tn_hbm, tn_v)
    pltpu.sync_copy(cx_hbm.at[pl.ds(base, DPW)], cxl_v)
    pltpu.sync_copy(cy_hbm.at[pl.ds(base, DPW)], cyl_v)

    for g in range(DPW // 16):
        cxv = cxl_v[pl.ds(g * 16, 16)]
        cyv = cyl_v[pl.ds(g * 16, 16)]

        z = jnp.zeros((16,), jnp.float32)

        @plsc.parallel_loop(0, E // CHUNK, carry=(z, z, z), unroll=2)
        def chunk_step(c, carry, cxv=cxv, cyv=cyv):
            s0, s1, cn = carry
            ebase = jnp.zeros((16,), jnp.int32) + c * CHUNK
            for j in range(CHUNK):
                idx = ebase + j
                exs = plsc.load_gather(ex_v, [idx])
                eys = plsc.load_gather(ey_v, [idx])
                dxf = jnp.clip(exs - cxv, -50.0, 50.0)
                dyf = jnp.clip(eys - cyv, -50.0, 50.0)
                dxi = dxf.astype(jnp.int32)
                dyi = dyf.astype(jnp.int32)
                f = dxi * 101 + (dyi + (50 * 101 + 50))
                s0 = s0 + plsc.load_gather(t0_v, [f])
                s1 = s1 + plsc.load_gather(t1_v, [f])
                cn = cn + plsc.load_gather(tn_v, [f])
            return (s0, s1, cn)

        s0, s1, cn = chunk_step
        o0_v[pl.ds(g * 16, 16)] = s0
        o1_v[pl.ds(g * 16, 16)] = s1
        od_v[pl.ds(g * 16, 16)] = jnp.where(cn >= THRESHOLD, 1.0, 0.0).astype(jnp.float32)

    pltpu.sync_copy(o0_v, s0_hbm.at[pl.ds(base, DPW)])
    pltpu.sync_copy(o1_v, s1_hbm.at[pl.ds(base, DPW)])
    pltpu.sync_copy(od_v, dec_hbm.at[pl.ds(base, DPW)])


_sc_gather = pl.kernel(
    _sc_body,
    out_type=(
        jax.ShapeDtypeStruct((N,), jnp.float32),
        jax.ShapeDtypeStruct((N,), jnp.float32),
        jax.ShapeDtypeStruct((N,), jnp.float32),
    ),
    mesh=plsc.VectorSubcoreMesh(core_axis_name="c", subcore_axis_name="s"),
    compiler_params=pltpu.CompilerParams(needs_layout_passes=False),
    scratch_types=[
        pltpu.VMEM((E,), jnp.float32),
        pltpu.VMEM((E,), jnp.float32),
        pltpu.VMEM((TABP,), jnp.float32),
        pltpu.VMEM((TABP,), jnp.float32),
        pltpu.VMEM((TABP,), jnp.float32),
        pltpu.VMEM((DPW,), jnp.float32),
        pltpu.VMEM((DPW,), jnp.float32),
        pltpu.VMEM((DPW,), jnp.float32),
        pltpu.VMEM((DPW,), jnp.float32),
        pltpu.VMEM((DPW,), jnp.float32),
    ],
)

BR = 256


def _tc_pair_body(cc, mask, pd, cxr, cyr, corr, cdx, cdy):
    cyrow = cc[:, 0:1]
    cxrow = cc[:, 1:2]
    dxc = cxr[0:1, :] - cxrow
    dyc = cyr[0:1, :] - cyrow
    m = mask[...]
    sdx = dxc * m
    sdy = dyc * m
    p = pd[...]
    radi = sdx * sdx + sdy * sdy - p * p
    stx = jnp.sum(4.0 * dxc * radi, axis=1, keepdims=True)
    sty = jnp.sum(4.0 * dyc * radi, axis=1, keepdims=True)
    cdx[...] = corr[...] * stx
    cdy[...] = corr[...] * sty


_tc_pair = pl.pallas_call(
    _tc_pair_body,
    grid=(N // BR,),
    in_specs=[
        pl.BlockSpec((BR, 2), lambda i: (i, 0)),
        pl.BlockSpec((BR, N), lambda i: (i, 0)),
        pl.BlockSpec((BR, N), lambda i: (i, 0)),
        pl.BlockSpec((1, N), lambda i: (0, 0)),
        pl.BlockSpec((1, N), lambda i: (0, 0)),
        pl.BlockSpec((BR, 1), lambda i: (i, 0)),
    ],
    out_specs=[
        pl.BlockSpec((BR, 1), lambda i: (i, 0)),
        pl.BlockSpec((BR, 1), lambda i: (i, 0)),
    ],
    out_shape=[
        jax.ShapeDtypeStruct((N, 1), jnp.float32),
        jax.ShapeDtypeStruct((N, 1), jnp.float32),
    ],
)


def _tc_combine_body(cc, s0, s1, dec, cdx, cdy, ny, nx):
    d = dec[...]
    ux = jnp.clip(s0[...], -400.0, 400.0)
    uy = jnp.clip(s1[...], -400.0, 400.0)
    nx[...] = cc[:, 1:2] - 200 * 1.5e-05 * d * (ux - 1.0 * 2.5e-07 * cdx[...])
    ny[...] = cc[:, 0:1] - 200 * 1.5e-05 * d * (uy - 1.0 * 2.5e-07 * cdy[...])


_tc_combine = pl.pallas_call(
    _tc_combine_body,
    out_shape=[
        jax.ShapeDtypeStruct((N, 1), jnp.float32),
        jax.ShapeDtypeStruct((N, 1), jnp.float32),
    ],
)


@jax.jit
def kernel(events_x, events_y, calib_center, precompute_grid,
           pairwise_dists_mask, pairwise_dists, correction):
    exf = events_x.astype(jnp.float32)
    eyf = events_y.astype(jnp.float32)
    cx = calib_center[:, 1]
    cy = calib_center[:, 0]
    g0 = precompute_grid[:, :, 0].reshape(-1)
    g1 = precompute_grid[:, :, 1].reshape(-1)
    gn = (g0 != 0).astype(jnp.float32) + (g1 != 0).astype(jnp.float32)
    pad = TABP - TAB
    t0 = jnp.pad(g0, (0, pad))
    t1 = jnp.pad(g1, (0, pad))
    tn = jnp.pad(gn, (0, pad))
    s0, s1, dec = _sc_gather(exf, eyf, cx, cy, t0, t1, tn)
    cdx, cdy = _tc_pair(calib_center, pairwise_dists_mask, pairwise_dists,
                        cx.reshape(1, N), cy.reshape(1, N),
                        correction.reshape(N, 1))
    ny, nx = _tc_combine(calib_center, s0.reshape(N, 1), s1.reshape(N, 1),
                         dec.reshape(N, 1), cdx, cdy)
    return jnp.concatenate([ny, nx], axis=1)


# speedup vs baseline: 1.1976x; 1.0885x over previous
import jax
import jax.numpy as jnp
from jax import lax
from jax.experimental import pallas as pl
from jax.experimental.pallas import tpu as pltpu
from jax.experimental.pallas import tpu_sc as plsc

N = 2048
E = 4096
NC = 2
NS = 16
NW = NC * NS
DPW = N // NW
TAB = 101 * 101
TABP = 10208
THRESHOLD = 10.0
CHUNK = 4


def _sc_body(exi_hbm, eyi_hbm, cc_hbm, t0_hbm, t1_hbm, tn_hbm,
             s0_hbm, s1_hbm, dec_hbm,
             exi_v, eyi_v, ex_v, ey_v, t0_v, t1_v, tn_v, ccl_v,
             o0_v, o1_v, od_v):
    cid = lax.axis_index("c")
    sid = lax.axis_index("s")
    wid = sid * NC + cid
    base = wid * DPW

    pltpu.sync_copy(exi_hbm, exi_v)
    pltpu.sync_copy(eyi_hbm, eyi_v)
    pltpu.sync_copy(t0_hbm, t0_v)
    pltpu.sync_copy(t1_hbm, t1_v)
    pltpu.sync_copy(tn_hbm, tn_v)
    pltpu.sync_copy(cc_hbm.at[pl.ds(base, DPW), :], ccl_v)

    @plsc.parallel_loop(0, E // 16)
    def cvt_step(i):
        ex_v[pl.ds(i * 16, 16)] = exi_v[pl.ds(i * 16, 16)].astype(jnp.float32)
        ey_v[pl.ds(i * 16, 16)] = eyi_v[pl.ds(i * 16, 16)].astype(jnp.float32)

    lanes = lax.broadcasted_iota(jnp.int32, (16,), 0)
    ones = jnp.ones((16,), jnp.int32)
    zeros = jnp.zeros((16,), jnp.int32)

    for g in range(DPW // 16):
        rows = lanes + (g * 16)
        cxv = plsc.load_gather(ccl_v, [rows, ones])
        cyv = plsc.load_gather(ccl_v, [rows, zeros])

        z = jnp.zeros((16,), jnp.float32)

        @plsc.parallel_loop(0, E // CHUNK, carry=(z, z, z), unroll=2)
        def chunk_step(c, carry, cxv=cxv, cyv=cyv):
            s0, s1, cn = carry
            ebase = jnp.zeros((16,), jnp.int32) + c * CHUNK
            for j in range(CHUNK):
                idx = ebase + j
                exs = plsc.load_gather(ex_v, [idx])
                eys = plsc.load_gather(ey_v, [idx])
                dxf = jnp.clip(exs - cxv, -50.0, 50.0)
                dyf = jnp.clip(eys - cyv, -50.0, 50.0)
                dxi = dxf.astype(jnp.int32)
                dyi = dyf.astype(jnp.int32)
                f = dxi * 101 + (dyi + (50 * 101 + 50))
                s0 = s0 + plsc.load_gather(t0_v, [f])
                s1 = s1 + plsc.load_gather(t1_v, [f])
                cn = cn + plsc.load_gather(tn_v, [f])
            return (s0, s1, cn)

        s0, s1, cn = chunk_step
        o0_v[pl.ds(g * 16, 16)] = s0
        o1_v[pl.ds(g * 16, 16)] = s1
        od_v[pl.ds(g * 16, 16)] = jnp.where(cn >= THRESHOLD, 1.0, 0.0).astype(jnp.float32)

    pltpu.sync_copy(o0_v, s0_hbm.at[pl.ds(base, DPW)])
    pltpu.sync_copy(o1_v, s1_hbm.at[pl.ds(base, DPW)])
    pltpu.sync_copy(od_v, dec_hbm.at[pl.ds(base, DPW)])


_sc_gather = pl.kernel(
    _sc_body,
    out_type=(
        jax.ShapeDtypeStruct((N,), jnp.float32),
        jax.ShapeDtypeStruct((N,), jnp.float32),
        jax.ShapeDtypeStruct((N,), jnp.float32),
    ),
    mesh=plsc.VectorSubcoreMesh(core_axis_name="c", subcore_axis_name="s"),
    compiler_params=pltpu.CompilerParams(needs_layout_passes=False),
    scratch_types=[
        pltpu.VMEM((E,), jnp.int32),
        pltpu.VMEM((E,), jnp.int32),
        pltpu.VMEM((E,), jnp.float32),
        pltpu.VMEM((E,), jnp.float32),
        pltpu.VMEM((TABP,), jnp.float32),
        pltpu.VMEM((TABP,), jnp.float32),
        pltpu.VMEM((TABP,), jnp.float32),
        pltpu.VMEM((DPW, 2), jnp.float32),
        pltpu.VMEM((DPW,), jnp.float32),
        pltpu.VMEM((DPW,), jnp.float32),
        pltpu.VMEM((DPW,), jnp.float32),
    ],
)

BR = 256


def _tc_pair_body(cc, ccf, corr, mask, pd, cdx, cdy):
    cyrow = cc[:, 0:1]
    cxrow = cc[:, 1:2]
    cxr = jnp.transpose(ccf[:, 1:2])
    cyr = jnp.transpose(ccf[:, 0:1])
    dxc = cxr - cxrow
    dyc = cyr - cyrow
    m = mask[...]
    sdx = dxc * m
    sdy = dyc * m
    p = pd[...]
    radi = sdx * sdx + sdy * sdy - p * p
    stx = jnp.sum(4.0 * dxc * radi, axis=1, keepdims=True)
    sty = jnp.sum(4.0 * dyc * radi, axis=1, keepdims=True)
    cdx[...] = jnp.transpose(corr[...] * stx)
    cdy[...] = jnp.transpose(corr[...] * sty)


_tc_pair = pl.pallas_call(
    _tc_pair_body,
    grid=(N // BR,),
    in_specs=[
        pl.BlockSpec((BR, 2), lambda i: (i, 0)),
        pl.BlockSpec((N, 2), lambda i: (0, 0)),
        pl.BlockSpec((BR, 1), lambda i: (i, 0)),
        pl.BlockSpec((BR, N), lambda i: (i, 0)),
        pl.BlockSpec((BR, N), lambda i: (i, 0)),
    ],
    out_specs=[
        pl.BlockSpec((1, BR), lambda i: (0, i)),
        pl.BlockSpec((1, BR), lambda i: (0, i)),
    ],
    out_shape=[
        jax.ShapeDtypeStruct((1, N), jnp.float32),
        jax.ShapeDtypeStruct((1, N), jnp.float32),
    ],
)


def _tc_combine_body(ccf, s0, s1, dec, cdx, cdy, out):
    cxl = jnp.transpose(ccf[:, 1:2])
    cyl = jnp.transpose(ccf[:, 0:1])
    d = dec[...].reshape(1, N)
    ux = jnp.clip(s0[...].reshape(1, N), -400.0, 400.0)
    uy = jnp.clip(s1[...].reshape(1, N), -400.0, 400.0)
    nx = cxl - 200 * 1.5e-05 * d * (ux - 1.0 * 2.5e-07 * cdx[...])
    ny = cyl - 200 * 1.5e-05 * d * (uy - 1.0 * 2.5e-07 * cdy[...])
    out[...] = jnp.transpose(jnp.concatenate([ny, nx], axis=0))


_tc_combine = pl.pallas_call(
    _tc_combine_body,
    out_shape=jax.ShapeDtypeStruct((N, 2), jnp.float32),
)


@jax.jit
def kernel(events_x, events_y, calib_center, precompute_grid,
           pairwise_dists_mask, pairwise_dists, correction):
    g0 = precompute_grid[:, :, 0].reshape(-1)
    g1 = precompute_grid[:, :, 1].reshape(-1)
    gn = (g0 != 0).astype(jnp.float32) + (g1 != 0).astype(jnp.float32)
    pad = TABP - TAB
    t0 = jnp.pad(g0, (0, pad))
    t1 = jnp.pad(g1, (0, pad))
    tn = jnp.pad(gn, (0, pad))
    s0, s1, dec = _sc_gather(events_x.astype(jnp.int32),
                             events_y.astype(jnp.int32),
                             calib_center, t0, t1, tn)
    cdx, cdy = _tc_pair(calib_center, calib_center,
                        correction.reshape(N, 1),
                        pairwise_dists_mask, pairwise_dists)
    return _tc_combine(calib_center, s0, s1, dec, cdx, cdy)
